# baseline (speedup 1.0000x reference)
import functools

import jax
import jax.numpy as jnp
from jax import lax
from jax.experimental import pallas as pl
from jax.experimental.pallas import tpu as pltpu
from jax.experimental.pallas import tpu_sc as plsc

N = 10000
NP = 10240
DUMP = 10200
E1 = 320000
E2 = 640000
DI, DH, DO = 128, 128, 64

NC, NS, L = 2, 16, 16
NW = NC * NS
CHUNK = 128
C1 = (E1 + NW * CHUNK - 1) // (NW * CHUNK)
C2 = (E2 + NW * CHUNK - 1) // (NW * CHUNK)
E1P = NW * CHUNK * C1
E2P = NW * CHUNK * C2
ROWS_PER_TILE = NP // NS

_mesh = plsc.VectorSubcoreMesh(core_axis_name="c", subcore_axis_name="s")


def _worker_id():
    return lax.axis_index("c") * NS + lax.axis_index("s")


def _deg_body(dst_hbm, out_hbm, idx_v, ones_v, accum_sh):
    cid = lax.axis_index("c")
    sid = lax.axis_index("s")
    wid = _worker_id()
    pltpu.sync_copy(dst_hbm.at[wid], idx_v)

    def _zero(i, _):
        ones_v[i] = jnp.zeros((L,), jnp.float32)
        return 0
    lax.fori_loop(0, CHUNK, _zero, 0)
    base = sid * ROWS_PER_TILE
    for k in range(ROWS_PER_TILE // CHUNK):
        pltpu.sync_copy(ones_v, accum_sh.at[pl.ds(base + k * CHUNK, CHUNK)])
    plsc.subcore_barrier()

    def _one(i, _):
        ones_v[i] = jnp.ones((L,), jnp.float32)
        return 0
    lax.fori_loop(0, CHUNK, _one, 0)

    def _chunk(j, _):
        pltpu.sync_copy(ones_v, accum_sh.at[idx_v.at[j]], add=True)
        return 0
    lax.fori_loop(0, C1, _chunk, 0)
    plsc.subcore_barrier()
    pltpu.sync_copy(accum_sh.at[pl.ds(base, ROWS_PER_TILE)],
                    out_hbm.at[cid, pl.ds(base, ROWS_PER_TILE)])


_deg_kernel = pl.kernel(
    _deg_body,
    out_type=jax.ShapeDtypeStruct((NC, NP, L), jnp.float32),
    mesh=_mesh,
    scratch_types=[
        pltpu.VMEM((C1, CHUNK), jnp.int32),
        pltpu.VMEM((CHUNK, L), jnp.float32),
        pltpu.VMEM_SHARED((NP, L), jnp.float32),
    ],
)


def _agg_body(src_hbm, dst_hbm, feat_hbm, out_hbm,
              sidx_v, didx_v, rows_v, accum_sh, sem, *, d):
    cid = lax.axis_index("c")
    sid = lax.axis_index("s")
    wid = _worker_id()
    pltpu.sync_copy(src_hbm.at[wid], sidx_v)
    pltpu.sync_copy(dst_hbm.at[wid], didx_v)

    def _zero(i, _):
        for j in range(d // L):
            rows_v[i, pl.ds(j * L, L)] = jnp.zeros((L,), jnp.float32)
        return 0
    lax.fori_loop(0, CHUNK, _zero, 0)
    base = sid * ROWS_PER_TILE
    for k in range(ROWS_PER_TILE // CHUNK):
        pltpu.sync_copy(rows_v, accum_sh.at[pl.ds(base + k * CHUNK, CHUNK)])
    plsc.subcore_barrier()

    def _chunk(j, _):
        pltpu.async_copy(feat_hbm.at[sidx_v.at[j]], rows_v, sem).wait()
        pltpu.sync_copy(rows_v, accum_sh.at[didx_v.at[j]], add=True)
        return 0
    lax.fori_loop(0, C1, _chunk, 0)
    plsc.subcore_barrier()
    pltpu.sync_copy(accum_sh.at[pl.ds(base, ROWS_PER_TILE)],
                    out_hbm.at[cid, pl.ds(base, ROWS_PER_TILE)])


def _make_agg_kernel(d):
    return pl.kernel(
        functools.partial(_agg_body, d=d),
        out_type=jax.ShapeDtypeStruct((NC, NP, d), jnp.float32),
        mesh=_mesh,
        scratch_types=[
            pltpu.VMEM((C1, CHUNK), jnp.int32),
            pltpu.VMEM((C1, CHUNK), jnp.int32),
            pltpu.VMEM((CHUNK, d), jnp.float32),
            pltpu.VMEM_SHARED((NP, d), jnp.float32),
            pltpu.SemaphoreType.DMA,
        ],
    )


_agg128 = _make_agg_kernel(DH)


def _decode_body(src_hbm, dst_hbm, z_hbm, out_hbm,
                 sidx_v, didx_v, srows_v, drows_v, pt_v, s1, s2):
    wid = _worker_id()
    pltpu.sync_copy(src_hbm.at[wid], sidx_v)
    pltpu.sync_copy(dst_hbm.at[wid], didx_v)

    def _chunk(c, _):
        g1 = pltpu.async_copy(z_hbm.at[sidx_v.at[c]], srows_v, s1)
        g2 = pltpu.async_copy(z_hbm.at[didx_v.at[c]], drows_v, s2)
        g1.wait()
        g2.wait()

        def _group(g, _):
            for l in range(L):
                e = g * L + l
                acc = srows_v[e, pl.ds(0, L)] * drows_v[e, pl.ds(0, L)]
                for j in range(1, DO // L):
                    acc = acc + (srows_v[e, pl.ds(j * L, L)] *
                                 drows_v[e, pl.ds(j * L, L)])
                pt_v[2 * g + l // 8, pl.ds((l % 8) * L, L)] = acc
            return 0
        lax.fori_loop(0, CHUNK // L, _group, 0)
        pltpu.sync_copy(pt_v, out_hbm.at[wid, c])
        return 0
    lax.fori_loop(0, C2, _chunk, 0)


_decode_kernel = pl.kernel(
    _decode_body,
    out_type=jax.ShapeDtypeStruct((NW, C2, L, CHUNK), jnp.float32),
    mesh=_mesh,
    scratch_types=[
        pltpu.VMEM((C2, CHUNK), jnp.int32),
        pltpu.VMEM((C2, CHUNK), jnp.int32),
        pltpu.VMEM((CHUNK, DH), jnp.float32),
        pltpu.VMEM((CHUNK, DH), jnp.float32),
        pltpu.VMEM((L, CHUNK), jnp.float32),
        pltpu.SemaphoreType.DMA,
        pltpu.SemaphoreType.DMA,
    ],
)


def _dinv_block(dp_ref):
    deg = dp_ref[0, :, 0:1] + dp_ref[1, :, 0:1] + 1.0
    return lax.rsqrt(deg)


def _stage1_body(x_ref, w1_ref, dp_ref, h1p_ref):
    dinv = _dinv_block(dp_ref)
    h1p_ref[...] = jnp.dot(x_ref[...], w1_ref[...],
                           preferred_element_type=jnp.float32) * dinv


def _stage2_body(a_ref, h1p_ref, dp_ref, w2_ref, b1_ref, h2p_ref):
    dinv = _dinv_block(dp_ref)
    z1 = jnp.maximum(
        dinv * (a_ref[0] + a_ref[1] + h1p_ref[...]) + b1_ref[...], 0.0)
    h2p_ref[...] = jnp.dot(z1, w2_ref[...],
                           preferred_element_type=jnp.float32) * dinv


def _stage3_body(a_ref, h2p_ref, dp_ref, b2_ref, z2_ref):
    dinv = _dinv_block(dp_ref)
    z2_ref[...] = dinv * (a_ref[0] + a_ref[1] + h2p_ref[...]) + b2_ref[...]


_BLK = 256
_GRID = NP // _BLK


def _rows(i):
    return (i, 0)


def _rep(i):
    return (0, 0)


def _3d_map(i):
    return (0, i, 0)


_stage1 = pl.pallas_call(
    _stage1_body,
    grid=(_GRID,),
    in_specs=[
        pl.BlockSpec((_BLK, DI), _rows),
        pl.BlockSpec((DI, DH), _rep),
        pl.BlockSpec((NC, _BLK, L), _3d_map),
    ],
    out_specs=pl.BlockSpec((_BLK, DH), _rows),
    out_shape=jax.ShapeDtypeStruct((NP, DH), jnp.float32),
)

_stage2 = pl.pallas_call(
    _stage2_body,
    grid=(_GRID,),
    in_specs=[
        pl.BlockSpec((NC, _BLK, DH), _3d_map),
        pl.BlockSpec((_BLK, DH), _rows),
        pl.BlockSpec((NC, _BLK, L), _3d_map),
        pl.BlockSpec((DH, DH), _rep),
        pl.BlockSpec((1, DH), _rep),
    ],
    out_specs=pl.BlockSpec((_BLK, DH), _rows),
    out_shape=jax.ShapeDtypeStruct((NP, DH), jnp.float32),
)

_stage3 = pl.pallas_call(
    _stage3_body,
    grid=(_GRID,),
    in_specs=[
        pl.BlockSpec((NC, _BLK, DH), _3d_map),
        pl.BlockSpec((_BLK, DH), _rows),
        pl.BlockSpec((NC, _BLK, L), _3d_map),
        pl.BlockSpec((1, DH), _rep),
    ],
    out_specs=pl.BlockSpec((_BLK, DH), _rows),
    out_shape=jax.ShapeDtypeStruct((NP, DH), jnp.float32),
)


def _reduce_body(x_ref, a_ref, y_ref):
    y_ref[...] = jnp.dot(x_ref[...], a_ref[...],
                         preferred_element_type=jnp.float32)


_RBLK = 512
_reduce = pl.pallas_call(
    _reduce_body,
    grid=(NW * C2 * L // _RBLK,),
    in_specs=[
        pl.BlockSpec((_RBLK, CHUNK), _rows),
        pl.BlockSpec((CHUNK, 8), _rep),
    ],
    out_specs=pl.BlockSpec((_RBLK, 8), _rows),
    out_shape=jax.ShapeDtypeStruct((NW * C2 * L, 8), jnp.float32),
)


def _pad_edges(idx, epad):
    pad = jnp.full((2, epad - idx.shape[1]), DUMP, jnp.int32)
    return jnp.concatenate([idx.astype(jnp.int32), pad], axis=1)


@jax.jit
def kernel(x, pos_edge_index, neg_edge_index, W1, b1, W2, b2):
    e1 = _pad_edges(pos_edge_index, E1P)
    src1 = e1[0].reshape(NW, C1, CHUNK)
    dst1 = e1[1].reshape(NW, C1, CHUNK)
    e2 = _pad_edges(
        jnp.concatenate([pos_edge_index, neg_edge_index], axis=1), E2P)
    src2 = e2[0].reshape(NW, C2, CHUNK)
    dst2 = e2[1].reshape(NW, C2, CHUNK)

    xp = jnp.pad(x, ((0, NP - N), (0, 0)))

    w2p = jnp.pad(W2, ((0, 0), (0, DH - DO)))
    b2p = jnp.pad(b2, (0, DH - DO)).reshape(1, DH)

    dp = _deg_kernel(dst1)
    h1p = _stage1(xp, W1, dp)
    a1 = _agg128(src1, dst1, h1p)
    h2p = _stage2(a1, h1p, dp, w2p, b1.reshape(1, DH))
    a2 = _agg128(src1, dst1, h2p)
    z2 = _stage3(a2, h2p, dp, b2p)
    parts = _decode_kernel(src2, dst2, z2)
    sel = jnp.repeat(jnp.eye(8, dtype=jnp.float32), L, axis=0)
    logits = _reduce(parts.reshape(NW * C2 * L, CHUNK), sel)
    return logits.reshape(-1)[:E2]


# speedup vs baseline: 1.0011x; 1.0011x over previous
import functools

import jax
import jax.numpy as jnp
from jax import lax
from jax.experimental import pallas as pl
from jax.experimental.pallas import tpu as pltpu
from jax.experimental.pallas import tpu_sc as plsc

N = 10000
NP = 10240
DUMP = 10200
E1 = 320000
E2 = 640000
DI, DH, DO = 128, 128, 64

NC, NS, L = 2, 16, 16
NW = NC * NS
CHUNK = 128
C1 = (E1 + NW * CHUNK - 1) // (NW * CHUNK)
C2 = (E2 + NW * CHUNK - 1) // (NW * CHUNK)
E1P = NW * CHUNK * C1
E2P = NW * CHUNK * C2
ROWS_PER_TILE = NP // NS

_mesh = plsc.VectorSubcoreMesh(core_axis_name="c", subcore_axis_name="s")


def _worker_id():
    return lax.axis_index("c") * NS + lax.axis_index("s")


def _deg_body(dst_hbm, out_hbm, idx_v, ones_v, accum_sh):
    cid = lax.axis_index("c")
    sid = lax.axis_index("s")
    wid = _worker_id()
    pltpu.sync_copy(dst_hbm.at[wid], idx_v)

    def _zero(i, _):
        ones_v[i] = jnp.zeros((L,), jnp.float32)
        return 0
    lax.fori_loop(0, CHUNK, _zero, 0)
    base = sid * ROWS_PER_TILE
    for k in range(ROWS_PER_TILE // CHUNK):
        pltpu.sync_copy(ones_v, accum_sh.at[pl.ds(base + k * CHUNK, CHUNK)])
    plsc.subcore_barrier()

    def _one(i, _):
        ones_v[i] = jnp.ones((L,), jnp.float32)
        return 0
    lax.fori_loop(0, CHUNK, _one, 0)

    def _chunk(j, _):
        pltpu.sync_copy(ones_v, accum_sh.at[idx_v.at[j]], add=True)
        return 0
    lax.fori_loop(0, C1, _chunk, 0)
    plsc.subcore_barrier()
    pltpu.sync_copy(accum_sh.at[pl.ds(base, ROWS_PER_TILE)],
                    out_hbm.at[cid, pl.ds(base, ROWS_PER_TILE)])


_deg_kernel = pl.kernel(
    _deg_body,
    out_type=jax.ShapeDtypeStruct((NC, NP, L), jnp.float32),
    mesh=_mesh,
    scratch_types=[
        pltpu.VMEM((C1, CHUNK), jnp.int32),
        pltpu.VMEM((CHUNK, L), jnp.float32),
        pltpu.VMEM_SHARED((NP, L), jnp.float32),
    ],
)


def _agg_body(src_hbm, dst_hbm, feat_hbm, out_hbm,
              sidx_v, didx_v, rows_v, accum_sh, sem, *, d):
    cid = lax.axis_index("c")
    sid = lax.axis_index("s")
    wid = _worker_id()
    pltpu.sync_copy(src_hbm.at[wid], sidx_v)
    pltpu.sync_copy(dst_hbm.at[wid], didx_v)

    def _zero(i, _):
        for j in range(d // L):
            rows_v[i, pl.ds(j * L, L)] = jnp.zeros((L,), jnp.float32)
        return 0
    lax.fori_loop(0, CHUNK, _zero, 0)
    base = sid * ROWS_PER_TILE
    for k in range(ROWS_PER_TILE // CHUNK):
        pltpu.sync_copy(rows_v, accum_sh.at[pl.ds(base + k * CHUNK, CHUNK)])
    plsc.subcore_barrier()

    def _chunk(j, _):
        pltpu.async_copy(feat_hbm.at[sidx_v.at[j]], rows_v, sem).wait()
        pltpu.sync_copy(rows_v, accum_sh.at[didx_v.at[j]], add=True)
        return 0
    lax.fori_loop(0, C1, _chunk, 0)
    plsc.subcore_barrier()
    pltpu.sync_copy(accum_sh.at[pl.ds(base, ROWS_PER_TILE)],
                    out_hbm.at[cid, pl.ds(base, ROWS_PER_TILE)])


def _make_agg_kernel(d):
    return pl.kernel(
        functools.partial(_agg_body, d=d),
        out_type=jax.ShapeDtypeStruct((NC, NP, d), jnp.float32),
        mesh=_mesh,
        scratch_types=[
            pltpu.VMEM((C1, CHUNK), jnp.int32),
            pltpu.VMEM((C1, CHUNK), jnp.int32),
            pltpu.VMEM((CHUNK, d), jnp.float32),
            pltpu.VMEM_SHARED((NP, d), jnp.float32),
            pltpu.SemaphoreType.DMA,
        ],
    )


_agg128 = _make_agg_kernel(DH)


def _decode_body(src_hbm, dst_hbm, z_hbm, out_hbm,
                 sidx_v, didx_v, srows_v, drows_v, pt_v, s1, s2):
    wid = _worker_id()
    pltpu.sync_copy(src_hbm.at[wid], sidx_v)
    pltpu.sync_copy(dst_hbm.at[wid], didx_v)

    def _chunk(c, _):
        g1 = pltpu.async_copy(z_hbm.at[sidx_v.at[c]], srows_v, s1)
        g2 = pltpu.async_copy(z_hbm.at[didx_v.at[c]], drows_v, s2)
        g1.wait()
        g2.wait()

        def _group(g, _):
            for l in range(L):
                e = g * L + l
                acc = srows_v[e, pl.ds(0, L)] * drows_v[e, pl.ds(0, L)]
                for j in range(1, DO // L):
                    acc = acc + (srows_v[e, pl.ds(j * L, L)] *
                                 drows_v[e, pl.ds(j * L, L)])
                pt_v[2 * g + l // 8, pl.ds((l % 8) * L, L)] = acc
            return 0
        lax.fori_loop(0, CHUNK // L, _group, 0)
        pltpu.sync_copy(pt_v, out_hbm.at[pl.ds((wid * C2 + c) * L, L)])
        return 0
    lax.fori_loop(0, C2, _chunk, 0)


_decode_kernel = pl.kernel(
    _decode_body,
    out_type=jax.ShapeDtypeStruct((NW * C2 * L, CHUNK), jnp.float32),
    mesh=_mesh,
    scratch_types=[
        pltpu.VMEM((C2, CHUNK), jnp.int32),
        pltpu.VMEM((C2, CHUNK), jnp.int32),
        pltpu.VMEM((CHUNK, DH), jnp.float32),
        pltpu.VMEM((CHUNK, DH), jnp.float32),
        pltpu.VMEM((L, CHUNK), jnp.float32),
        pltpu.SemaphoreType.DMA,
        pltpu.SemaphoreType.DMA,
    ],
)


def _dinv_block(dp_ref):
    deg = dp_ref[0, :, 0:1] + dp_ref[1, :, 0:1] + 1.0
    return lax.rsqrt(deg)


def _stage1_body(x_ref, w1_ref, dp_ref, h1p_ref):
    dinv = _dinv_block(dp_ref)
    h1p_ref[...] = jnp.dot(x_ref[...], w1_ref[...],
                           preferred_element_type=jnp.float32) * dinv


def _stage2_body(a_ref, h1p_ref, dp_ref, w2_ref, b1_ref, h2p_ref):
    dinv = _dinv_block(dp_ref)
    z1 = jnp.maximum(
        dinv * (a_ref[0] + a_ref[1] + h1p_ref[...]) + b1_ref[...], 0.0)
    h2p_ref[...] = jnp.dot(z1, w2_ref[...],
                           preferred_element_type=jnp.float32) * dinv


def _stage3_body(a_ref, h2p_ref, dp_ref, b2_ref, z2_ref):
    dinv = _dinv_block(dp_ref)
    z2_ref[...] = dinv * (a_ref[0] + a_ref[1] + h2p_ref[...]) + b2_ref[...]


_BLK = 256
_GRID = NP // _BLK


def _rows(i):
    return (i, 0)


def _rep(i):
    return (0, 0)


def _3d_map(i):
    return (0, i, 0)


_stage1 = pl.pallas_call(
    _stage1_body,
    grid=(_GRID,),
    in_specs=[
        pl.BlockSpec((_BLK, DI), _rows),
        pl.BlockSpec((DI, DH), _rep),
        pl.BlockSpec((NC, _BLK, L), _3d_map),
    ],
    out_specs=pl.BlockSpec((_BLK, DH), _rows),
    out_shape=jax.ShapeDtypeStruct((NP, DH), jnp.float32),
)

_stage2 = pl.pallas_call(
    _stage2_body,
    grid=(_GRID,),
    in_specs=[
        pl.BlockSpec((NC, _BLK, DH), _3d_map),
        pl.BlockSpec((_BLK, DH), _rows),
        pl.BlockSpec((NC, _BLK, L), _3d_map),
        pl.BlockSpec((DH, DH), _rep),
        pl.BlockSpec((1, DH), _rep),
    ],
    out_specs=pl.BlockSpec((_BLK, DH), _rows),
    out_shape=jax.ShapeDtypeStruct((NP, DH), jnp.float32),
)

_stage3 = pl.pallas_call(
    _stage3_body,
    grid=(_GRID,),
    in_specs=[
        pl.BlockSpec((NC, _BLK, DH), _3d_map),
        pl.BlockSpec((_BLK, DH), _rows),
        pl.BlockSpec((NC, _BLK, L), _3d_map),
        pl.BlockSpec((1, DH), _rep),
    ],
    out_specs=pl.BlockSpec((_BLK, DH), _rows),
    out_shape=jax.ShapeDtypeStruct((NP, DH), jnp.float32),
)


def _reduce_body(x_ref, a_ref, y_ref):
    y_ref[...] = jnp.dot(x_ref[...], a_ref[...],
                         preferred_element_type=jnp.float32)


_RBLK = 512
_reduce = pl.pallas_call(
    _reduce_body,
    grid=(NW * C2 * L // _RBLK,),
    in_specs=[
        pl.BlockSpec((_RBLK, CHUNK), _rows),
        pl.BlockSpec((CHUNK, 8), _rep),
    ],
    out_specs=pl.BlockSpec((_RBLK, 8), _rows),
    out_shape=jax.ShapeDtypeStruct((NW * C2 * L, 8), jnp.float32),
)


def _pad_edges(idx, epad):
    pad = jnp.full((2, epad - idx.shape[1]), DUMP, jnp.int32)
    return jnp.concatenate([idx.astype(jnp.int32), pad], axis=1)


@jax.jit
def kernel(x, pos_edge_index, neg_edge_index, W1, b1, W2, b2):
    e1 = _pad_edges(pos_edge_index, E1P)
    src1 = e1[0].reshape(NW, C1, CHUNK)
    dst1 = e1[1].reshape(NW, C1, CHUNK)
    e2 = _pad_edges(
        jnp.concatenate([pos_edge_index, neg_edge_index], axis=1), E2P)
    src2 = e2[0].reshape(NW, C2, CHUNK)
    dst2 = e2[1].reshape(NW, C2, CHUNK)

    xp = jnp.pad(x, ((0, NP - N), (0, 0)))

    w2p = jnp.pad(W2, ((0, 0), (0, DH - DO)))
    b2p = jnp.pad(b2, (0, DH - DO)).reshape(1, DH)

    dp = _deg_kernel(dst1)
    h1p = _stage1(xp, W1, dp)
    a1 = _agg128(src1, dst1, h1p)
    h2p = _stage2(a1, h1p, dp, w2p, b1.reshape(1, DH))
    a2 = _agg128(src1, dst1, h2p)
    z2 = _stage3(a2, h2p, dp, b2p)
    parts = _decode_kernel(src2, dst2, z2)
    sel = jnp.repeat(jnp.eye(8, dtype=jnp.float32), L, axis=0)
    logits = _reduce(parts, sel)
    return logits.reshape(-1)[:E2]


# speedup vs baseline: 1.1523x; 1.1511x over previous
import functools

import jax
import jax.numpy as jnp
from jax import lax
from jax.experimental import pallas as pl
from jax.experimental.pallas import tpu as pltpu
from jax.experimental.pallas import tpu_sc as plsc

N = 10000
NP = 10240
DUMP = 10200
E1 = 320000
E2 = 640000
DI, DH, DO = 128, 128, 64

NC, NS, L = 2, 16, 16
NW = NC * NS
CH1 = 64
CHUNK = 128
C1 = (E1 + NW * CH1 - 1) // (NW * CH1)
C2 = (E2 + NW * CHUNK - 1) // (NW * CHUNK)
E1P = NW * CH1 * C1
E2P = NW * CHUNK * C2
ROWS_PER_TILE = NP // NS

_mesh = plsc.VectorSubcoreMesh(core_axis_name="c", subcore_axis_name="s")


def _worker_id():
    return lax.axis_index("c") * NS + lax.axis_index("s")


def _deg_body(dst_hbm, out_hbm, idx_v, ones_v, accum_sh):
    cid = lax.axis_index("c")
    sid = lax.axis_index("s")
    wid = _worker_id()
    pltpu.sync_copy(dst_hbm.at[wid], idx_v)

    def _zero(i, _):
        ones_v[i] = jnp.zeros((L,), jnp.float32)
        return 0
    lax.fori_loop(0, CH1, _zero, 0)
    base = sid * ROWS_PER_TILE
    for k in range(ROWS_PER_TILE // CH1):
        pltpu.sync_copy(ones_v, accum_sh.at[pl.ds(base + k * CH1, CH1)])
    plsc.subcore_barrier()

    def _one(i, _):
        ones_v[i] = jnp.ones((L,), jnp.float32)
        return 0
    lax.fori_loop(0, CH1, _one, 0)

    def _chunk(j, _):
        pltpu.sync_copy(ones_v, accum_sh.at[idx_v.at[j]], add=True)
        return 0
    lax.fori_loop(0, C1, _chunk, 0)
    plsc.subcore_barrier()
    pltpu.sync_copy(accum_sh.at[pl.ds(base, ROWS_PER_TILE)],
                    out_hbm.at[cid, pl.ds(base, ROWS_PER_TILE)])


_deg_kernel = pl.kernel(
    _deg_body,
    out_type=jax.ShapeDtypeStruct((NC, NP, L), jnp.float32),
    mesh=_mesh,
    scratch_types=[
        pltpu.VMEM((C1, CH1), jnp.int32),
        pltpu.VMEM((CH1, L), jnp.float32),
        pltpu.VMEM_SHARED((NP, L), jnp.float32),
    ],
)


def _agg_body(src_hbm, dst_hbm, feat_hbm, out_hbm,
              sidx_v, didx_v, rows_a, accum_sh, sem_a, *, d):
    rows_b = rows_a
    cid = lax.axis_index("c")
    sid = lax.axis_index("s")
    wid = _worker_id()
    pltpu.sync_copy(src_hbm.at[wid], sidx_v)
    pltpu.sync_copy(dst_hbm.at[wid], didx_v)

    def _zero(i, _):
        for j in range(d // L):
            rows_a[i, pl.ds(j * L, L)] = jnp.zeros((L,), jnp.float32)
        return 0
    lax.fori_loop(0, CH1, _zero, 0)
    base = sid * ROWS_PER_TILE
    for k in range(ROWS_PER_TILE // CH1):
        pltpu.sync_copy(rows_a, accum_sh.at[pl.ds(base + k * CH1, CH1)])
    plsc.subcore_barrier()

    def _chunk(j, _):
        pltpu.async_copy(feat_hbm.at[sidx_v.at[j]], rows_b, sem_a).wait()
        pltpu.sync_copy(rows_b, accum_sh.at[didx_v.at[j]], add=True)
        return 0
    lax.fori_loop(0, C1, _chunk, 0)
    plsc.subcore_barrier()
    pltpu.sync_copy(accum_sh.at[pl.ds(base, ROWS_PER_TILE)],
                    out_hbm.at[cid, pl.ds(base, ROWS_PER_TILE)])


def _make_agg_kernel(d):
    return pl.kernel(
        functools.partial(_agg_body, d=d),
        out_type=jax.ShapeDtypeStruct((NC, NP, d), jnp.float32),
        mesh=_mesh,
        scratch_types=[
            pltpu.VMEM((C1, CH1), jnp.int32),
            pltpu.VMEM((C1, CH1), jnp.int32),
            pltpu.VMEM((CH1, d), jnp.float32),
            pltpu.VMEM_SHARED((NP, d), jnp.float32),
            pltpu.SemaphoreType.DMA,
        ],
    )


_agg128 = _make_agg_kernel(DH)


def _decode_body(src_hbm, dst_hbm, z_hbm, out_hbm,
                 sidx_v, didx_v, s_a, d_a, s_b, d_b, pt_a, pt_b,
                 ss_a, sd_a, ss_b, sd_b, so_a, so_b):
    wid = _worker_id()
    pltpu.sync_copy(src_hbm.at[wid], sidx_v)
    pltpu.sync_copy(dst_hbm.at[wid], didx_v)

    def _issue(t, srows, drows, ss, sd):
        pltpu.async_copy(z_hbm.at[sidx_v.at[t]], srows, ss)
        pltpu.async_copy(z_hbm.at[didx_v.at[t]], drows, sd)

    def _wait(srows, drows, ss, sd):
        pltpu.make_async_copy(z_hbm.at[sidx_v.at[0]], srows, ss).wait()
        pltpu.make_async_copy(z_hbm.at[sidx_v.at[0]], drows, sd).wait()

    def _compute(c, srows, drows, pt, so):
        def _group(g, _):
            for l in range(L):
                e = g * L + l
                acc = srows[e, pl.ds(0, L)] * drows[e, pl.ds(0, L)]
                for j in range(1, DO // L):
                    acc = acc + (srows[e, pl.ds(j * L, L)] *
                                 drows[e, pl.ds(j * L, L)])
                pt[2 * g + l // 8, pl.ds((l % 8) * L, L)] = acc
            return 0
        lax.fori_loop(0, CHUNK // L, _group, 0)
        pltpu.async_copy(pt, out_hbm.at[pl.ds((wid * C2 + c) * L, L)], so)

    def _wait_out(c0, pt, so):
        pltpu.make_async_copy(pt, out_hbm.at[pl.ds(wid * L, L)], so).wait()

    _issue(0, s_a, d_a, ss_a, sd_a)

    def _pair(p, _):
        t = 2 * p
        _wait(s_a, d_a, ss_a, sd_a)
        _issue(t + 1, s_b, d_b, ss_b, sd_b)

        @pl.when(p > 0)
        def _():
            _wait_out(t - 2, pt_a, so_a)
        _compute(t, s_a, d_a, pt_a, so_a)
        _wait(s_b, d_b, ss_b, sd_b)
        _issue(t + 2, s_a, d_a, ss_a, sd_a)

        @pl.when(p > 0)
        def _():
            _wait_out(t - 1, pt_b, so_b)
        _compute(t + 1, s_b, d_b, pt_b, so_b)
        return 0
    lax.fori_loop(0, (C2 - 1) // 2, _pair, 0)
    _wait(s_a, d_a, ss_a, sd_a)
    _wait_out(C2 - 3, pt_a, so_a)
    _compute(C2 - 1, s_a, d_a, pt_a, so_a)
    _wait_out(C2 - 2, pt_b, so_b)
    _wait_out(C2 - 1, pt_a, so_a)


_decode_kernel = pl.kernel(
    _decode_body,
    out_type=jax.ShapeDtypeStruct((NW * C2 * L, CHUNK), jnp.float32),
    mesh=_mesh,
    scratch_types=[
        pltpu.VMEM((C2, CHUNK), jnp.int32),
        pltpu.VMEM((C2, CHUNK), jnp.int32),
        pltpu.VMEM((CHUNK, DH), jnp.float32),
        pltpu.VMEM((CHUNK, DH), jnp.float32),
        pltpu.VMEM((CHUNK, DH), jnp.float32),
        pltpu.VMEM((CHUNK, DH), jnp.float32),
        pltpu.VMEM((L, CHUNK), jnp.float32),
        pltpu.VMEM((L, CHUNK), jnp.float32),
        pltpu.SemaphoreType.DMA,
        pltpu.SemaphoreType.DMA,
        pltpu.SemaphoreType.DMA,
        pltpu.SemaphoreType.DMA,
        pltpu.SemaphoreType.DMA,
        pltpu.SemaphoreType.DMA,
    ],
)


def _dinv_block(dp_ref):
    deg = dp_ref[0, :, 0:1] + dp_ref[1, :, 0:1] + 1.0
    return lax.rsqrt(deg)


def _stage1_body(x_ref, w1_ref, dp_ref, h1p_ref):
    dinv = _dinv_block(dp_ref)
    h1p_ref[...] = jnp.dot(x_ref[...], w1_ref[...],
                           preferred_element_type=jnp.float32) * dinv


def _stage2_body(a_ref, h1p_ref, dp_ref, w2_ref, b1_ref, h2p_ref):
    dinv = _dinv_block(dp_ref)
    z1 = jnp.maximum(
        dinv * (a_ref[0] + a_ref[1] + h1p_ref[...]) + b1_ref[...], 0.0)
    h2p_ref[...] = jnp.dot(z1, w2_ref[...],
                           preferred_element_type=jnp.float32) * dinv


def _stage3_body(a_ref, h2p_ref, dp_ref, b2_ref, z2_ref):
    dinv = _dinv_block(dp_ref)
    z2_ref[...] = dinv * (a_ref[0] + a_ref[1] + h2p_ref[...]) + b2_ref[...]


_BLK = 256
_GRID = NP // _BLK


def _rows(i):
    return (i, 0)


def _rep(i):
    return (0, 0)


def _3d_map(i):
    return (0, i, 0)


_stage1 = pl.pallas_call(
    _stage1_body,
    grid=(_GRID,),
    in_specs=[
        pl.BlockSpec((_BLK, DI), _rows),
        pl.BlockSpec((DI, DH), _rep),
        pl.BlockSpec((NC, _BLK, L), _3d_map),
    ],
    out_specs=pl.BlockSpec((_BLK, DH), _rows),
    out_shape=jax.ShapeDtypeStruct((NP, DH), jnp.float32),
)

_stage2 = pl.pallas_call(
    _stage2_body,
    grid=(_GRID,),
    in_specs=[
        pl.BlockSpec((NC, _BLK, DH), _3d_map),
        pl.BlockSpec((_BLK, DH), _rows),
        pl.BlockSpec((NC, _BLK, L), _3d_map),
        pl.BlockSpec((DH, DH), _rep),
        pl.BlockSpec((1, DH), _rep),
    ],
    out_specs=pl.BlockSpec((_BLK, DH), _rows),
    out_shape=jax.ShapeDtypeStruct((NP, DH), jnp.float32),
)

_stage3 = pl.pallas_call(
    _stage3_body,
    grid=(_GRID,),
    in_specs=[
        pl.BlockSpec((NC, _BLK, DH), _3d_map),
        pl.BlockSpec((_BLK, DH), _rows),
        pl.BlockSpec((NC, _BLK, L), _3d_map),
        pl.BlockSpec((1, DH), _rep),
    ],
    out_specs=pl.BlockSpec((_BLK, DH), _rows),
    out_shape=jax.ShapeDtypeStruct((NP, DH), jnp.float32),
)


def _reduce_body(x_ref, a_ref, y_ref):
    y_ref[...] = jnp.dot(x_ref[...], a_ref[...],
                         preferred_element_type=jnp.float32)


_RBLK = 512
_reduce = pl.pallas_call(
    _reduce_body,
    grid=(NW * C2 * L // _RBLK,),
    in_specs=[
        pl.BlockSpec((_RBLK, CHUNK), _rows),
        pl.BlockSpec((CHUNK, 8), _rep),
    ],
    out_specs=pl.BlockSpec((_RBLK, 8), _rows),
    out_shape=jax.ShapeDtypeStruct((NW * C2 * L, 8), jnp.float32),
)


def _pad_edges(idx, epad):
    pad = jnp.full((2, epad - idx.shape[1]), DUMP, jnp.int32)
    return jnp.concatenate([idx.astype(jnp.int32), pad], axis=1)


@jax.jit
def kernel(x, pos_edge_index, neg_edge_index, W1, b1, W2, b2):
    e1 = _pad_edges(pos_edge_index, E1P)
    src1 = e1[0].reshape(NW, C1, CH1)
    dst1 = e1[1].reshape(NW, C1, CH1)
    e2 = _pad_edges(
        jnp.concatenate([pos_edge_index, neg_edge_index], axis=1), E2P)
    src2 = e2[0].reshape(NW, C2, CHUNK)
    dst2 = e2[1].reshape(NW, C2, CHUNK)

    xp = jnp.pad(x, ((0, NP - N), (0, 0)))

    w2p = jnp.pad(W2, ((0, 0), (0, DH - DO)))
    b2p = jnp.pad(b2, (0, DH - DO)).reshape(1, DH)

    dp = _deg_kernel(dst1)
    h1p = _stage1(xp, W1, dp)
    a1 = _agg128(src1, dst1, h1p)
    h2p = _stage2(a1, h1p, dp, w2p, b1.reshape(1, DH))
    a2 = _agg128(src1, dst1, h2p)
    z2 = _stage3(a2, h2p, dp, b2p)
    parts = _decode_kernel(src2, dst2, z2)
    sel = jnp.repeat(jnp.eye(8, dtype=jnp.float32), L, axis=0)
    logits = _reduce(parts, sel)
    return logits.reshape(-1)[:E2]
